# initial kernel scaffold (unmeasured)
import jax
import jax.numpy as jnp
from jax import lax
from jax.experimental import pallas as pl
from jax.experimental.pallas import tpu as pltpu

N_DEV = 4


def kernel(A, B):
    m_per, k = A.shape
    _, n = B.shape

    a_bf = A.astype(jnp.bfloat16)
    b_bf = B.astype(jnp.bfloat16)

    def body(a_ref, b_ref, out_ref, comm_ref, send_sems, recv_sems):
        my = lax.axis_index("i")
        left = lax.rem(my + N_DEV - 1, N_DEV)
        right = lax.rem(my + 1, N_DEV)

        barrier_sem = pltpu.get_barrier_semaphore()
        for nbr in (left, right):
            pl.semaphore_signal(
                barrier_sem, inc=1,
                device_id=(nbr,), device_id_type=pl.DeviceIdType.MESH,
            )
        pl.semaphore_wait(barrier_sem, 2)

        def make_rdma(h):
            src = a_ref if h == 0 else comm_ref.at[h - 1]
            return pltpu.make_async_remote_copy(
                src_ref=src,
                dst_ref=comm_ref.at[h],
                send_sem=send_sems.at[h],
                recv_sem=recv_sems.at[h],
                device_id=(right,),
                device_id_type=pl.DeviceIdType.MESH,
            )

        rdmas = [make_rdma(h) for h in range(N_DEV - 1)]

        rdmas[0].start()
        out_ref[pl.ds(my * m_per, m_per), :] = jnp.dot(
            a_ref[...], b_ref[...], preferred_element_type=jnp.float32
        ).astype(jnp.bfloat16)

        for h in range(N_DEV - 1):
            rdmas[h].wait_recv()
            if h + 1 < N_DEV - 1:
                rdmas[h + 1].start()
            origin = lax.rem(my + N_DEV - h - 1, N_DEV)
            out_ref[pl.ds(origin * m_per, m_per), :] = jnp.dot(
                comm_ref[h], b_ref[...], preferred_element_type=jnp.float32
            ).astype(jnp.bfloat16)

        for h in range(N_DEV - 1):
            rdmas[h].wait_send()

    return pl.pallas_call(
        body,
        out_shape=jax.ShapeDtypeStruct((N_DEV * m_per, n), jnp.bfloat16),
        in_specs=[
            pl.BlockSpec(memory_space=pltpu.VMEM),
            pl.BlockSpec(memory_space=pltpu.VMEM),
        ],
        out_specs=pl.BlockSpec(memory_space=pltpu.VMEM),
        scratch_shapes=[
            pltpu.VMEM((N_DEV - 1, m_per, k), jnp.bfloat16),
            pltpu.SemaphoreType.DMA((N_DEV - 1,)),
            pltpu.SemaphoreType.DMA((N_DEV - 1,)),
        ],
        compiler_params=pltpu.CompilerParams(collective_id=0),
    )(a_bf, b_bf)


# baseline (device time: 197987 ns/iter reference)
import jax
import jax.numpy as jnp
from jax import lax
from jax.experimental import pallas as pl
from jax.experimental.pallas import tpu as pltpu

N_DEV = 4


def kernel(A, B):
    m_per, k = A.shape
    _, n = B.shape

    a_bf = A.astype(jnp.bfloat16)
    b_bf = B.astype(jnp.bfloat16)

    def body(a_ref, b_ref, out_ref, comm_ref, send_sems, recv_sems):
        my = lax.axis_index("i")
        left = lax.rem(my + N_DEV - 1, N_DEV)
        right = lax.rem(my + 1, N_DEV)

        barrier_sem = pltpu.get_barrier_semaphore()
        for nbr in (left, right):
            pl.semaphore_signal(
                barrier_sem, inc=1,
                device_id=(nbr,), device_id_type=pl.DeviceIdType.MESH,
            )
        pl.semaphore_wait(barrier_sem, 2)

        def make_rdma(h):
            src = a_ref if h == 0 else comm_ref.at[h - 1]
            return pltpu.make_async_remote_copy(
                src_ref=src,
                dst_ref=comm_ref.at[h],
                send_sem=send_sems.at[h],
                recv_sem=recv_sems.at[h],
                device_id=(right,),
                device_id_type=pl.DeviceIdType.MESH,
            )

        rdmas = [make_rdma(h) for h in range(N_DEV - 1)]

        rdmas[0].start()
        out_ref[pl.ds(my * m_per, m_per), :] = jnp.dot(
            a_ref[...], b_ref[...], preferred_element_type=jnp.float32
        ).astype(jnp.bfloat16)

        for h in range(N_DEV - 1):
            rdmas[h].wait_recv()
            if h + 1 < N_DEV - 1:
                rdmas[h + 1].start()
            origin = lax.rem(my + N_DEV - h - 1, N_DEV)
            out_ref[pl.ds(origin * m_per, m_per), :] = jnp.dot(
                comm_ref[h], b_ref[...], preferred_element_type=jnp.float32
            ).astype(jnp.bfloat16)

        for h in range(N_DEV - 1):
            rdmas[h].wait_send()

    return pl.pallas_call(
        body,
        out_shape=jax.ShapeDtypeStruct((N_DEV * m_per, n), jnp.bfloat16),
        in_specs=[
            pl.BlockSpec(memory_space=pltpu.VMEM),
            pl.BlockSpec(memory_space=pltpu.VMEM),
        ],
        out_specs=pl.BlockSpec(memory_space=pltpu.VMEM),
        scratch_shapes=[
            pltpu.VMEM((N_DEV - 1, m_per, k), jnp.bfloat16),
            pltpu.SemaphoreType.DMA((N_DEV - 1,)),
            pltpu.SemaphoreType.DMA((N_DEV - 1,)),
        ],
        compiler_params=pltpu.CompilerParams(
            collective_id=0,
            vmem_limit_bytes=100 * 1024 * 1024,
        ),
    )(a_bf, b_bf)


# device time: 138423 ns/iter; 1.4303x vs baseline; 1.4303x over previous
import jax
import jax.numpy as jnp
from jax import lax
from jax.experimental import pallas as pl
from jax.experimental.pallas import tpu as pltpu

N_DEV = 4


def kernel(A, B):
    m_per, k = A.shape
    _, n = B.shape
    half = m_per // 2

    a_bf = A.astype(jnp.bfloat16)
    b_bf = B.astype(jnp.bfloat16)

    def body(a_ref, b_ref, out_ref, cl_ref, cr_ref, co_ref, send_sems, recv_sems):
        my = lax.axis_index("i")
        left = lax.rem(my + N_DEV - 1, N_DEV)
        right = lax.rem(my + 1, N_DEV)
        opp = lax.rem(my + 2, N_DEV)

        barrier_sem = pltpu.get_barrier_semaphore()
        for nbr in (left, right):
            pl.semaphore_signal(
                barrier_sem, inc=1,
                device_id=(nbr,), device_id_type=pl.DeviceIdType.MESH,
            )
        pl.semaphore_wait(barrier_sem, 2)

        def rdma(src, dst, sem_idx, target):
            return pltpu.make_async_remote_copy(
                src_ref=src,
                dst_ref=dst,
                send_sem=send_sems.at[sem_idx],
                recv_sem=recv_sems.at[sem_idx],
                device_id=(target,),
                device_id_type=pl.DeviceIdType.MESH,
            )

        t1r = rdma(a_ref, cl_ref, 0, right)
        t1l = rdma(a_ref, cr_ref, 1, left)
        lo = pl.ds(0, half)
        hi = pl.ds(half, half)
        t2r = rdma(cl_ref.at[lo], co_ref.at[lo], 2, right)
        t2l = rdma(cr_ref.at[hi], co_ref.at[hi], 3, left)

        t1r.start()
        t1l.start()

        def mm(chunk):
            return jnp.dot(
                chunk, b_ref[...], preferred_element_type=jnp.float32
            ).astype(jnp.bfloat16)

        out_ref[pl.ds(my * m_per, m_per), :] = mm(a_ref[...])

        t1r.wait_recv()
        t2r.start()
        out_ref[pl.ds(left * m_per, m_per), :] = mm(cl_ref[...])

        t1l.wait_recv()
        t2l.start()
        out_ref[pl.ds(right * m_per, m_per), :] = mm(cr_ref[...])

        t2r.wait_recv()
        t2l.wait_recv()
        out_ref[pl.ds(opp * m_per, m_per), :] = mm(co_ref[...])

        for t in (t1r, t1l, t2r, t2l):
            t.wait_send()

    return pl.pallas_call(
        body,
        out_shape=jax.ShapeDtypeStruct((N_DEV * m_per, n), jnp.bfloat16),
        in_specs=[
            pl.BlockSpec(memory_space=pltpu.VMEM),
            pl.BlockSpec(memory_space=pltpu.VMEM),
        ],
        out_specs=pl.BlockSpec(memory_space=pltpu.VMEM),
        scratch_shapes=[
            pltpu.VMEM((m_per, k), jnp.bfloat16),
            pltpu.VMEM((m_per, k), jnp.bfloat16),
            pltpu.VMEM((m_per, k), jnp.bfloat16),
            pltpu.SemaphoreType.DMA((4,)),
            pltpu.SemaphoreType.DMA((4,)),
        ],
        compiler_params=pltpu.CompilerParams(
            collective_id=0,
            vmem_limit_bytes=100 * 1024 * 1024,
        ),
    )(a_bf, b_bf)


# device time: 126895 ns/iter; 1.5602x vs baseline; 1.0908x over previous
import jax
import jax.numpy as jnp
from jax import lax
from jax.experimental import pallas as pl
from jax.experimental.pallas import tpu as pltpu

N_DEV = 4


def kernel(A, B):
    m_per, k = A.shape
    _, n = B.shape
    half = m_per // 2

    a_bf = A.astype(jnp.bfloat16)
    b_bf = B.astype(jnp.bfloat16)

    def body(a_ref, b_ref, out_ref, cl_ref, cr_ref, co_ref, send_sems, recv_sems):
        my = lax.axis_index("i")
        left = lax.rem(my + N_DEV - 1, N_DEV)
        right = lax.rem(my + 1, N_DEV)
        opp = lax.rem(my + 2, N_DEV)

        barrier_sem = pltpu.get_barrier_semaphore()
        for nbr in (left, right):
            pl.semaphore_signal(
                barrier_sem, inc=1,
                device_id=(nbr,), device_id_type=pl.DeviceIdType.MESH,
            )
        pl.semaphore_wait(barrier_sem, 2)

        def rdma(src, dst, sem_idx, target):
            return pltpu.make_async_remote_copy(
                src_ref=src,
                dst_ref=dst,
                send_sem=send_sems.at[sem_idx],
                recv_sem=recv_sems.at[sem_idx],
                device_id=(target,),
                device_id_type=pl.DeviceIdType.MESH,
            )

        lo = pl.ds(0, half)
        hi = pl.ds(half, half)

        t1r_lo = rdma(a_ref.at[lo], cl_ref.at[lo], 0, right)
        t1r_hi = rdma(a_ref.at[hi], cl_ref.at[hi], 1, right)
        t1l_hi = rdma(a_ref.at[hi], cr_ref.at[hi], 2, left)
        t1l_lo = rdma(a_ref.at[lo], cr_ref.at[lo], 3, left)
        t2r = rdma(cl_ref.at[lo], co_ref.at[lo], 4, right)
        t2l = rdma(cr_ref.at[hi], co_ref.at[hi], 5, left)

        t1r_lo.start()
        t1l_hi.start()
        t1r_hi.start()
        t1l_lo.start()

        def mm(chunk, row0, nrows):
            out_ref[pl.ds(row0, nrows), :] = jnp.dot(
                chunk, b_ref[...], preferred_element_type=jnp.float32
            ).astype(jnp.bfloat16)

        mm(a_ref[...], my * m_per, m_per)

        t1r_lo.wait_recv()
        t2r.start()
        mm(cl_ref[lo, :], left * m_per, half)

        t1l_hi.wait_recv()
        t2l.start()
        mm(cr_ref[hi, :], right * m_per + half, half)

        t1r_hi.wait_recv()
        mm(cl_ref[hi, :], left * m_per + half, half)

        t1l_lo.wait_recv()
        mm(cr_ref[lo, :], right * m_per, half)

        t2r.wait_recv()
        mm(co_ref[lo, :], opp * m_per, half)

        t2l.wait_recv()
        mm(co_ref[hi, :], opp * m_per + half, half)

        for t in (t1r_lo, t1r_hi, t1l_hi, t1l_lo, t2r, t2l):
            t.wait_send()

    return pl.pallas_call(
        body,
        out_shape=jax.ShapeDtypeStruct((N_DEV * m_per, n), jnp.bfloat16),
        in_specs=[
            pl.BlockSpec(memory_space=pltpu.VMEM),
            pl.BlockSpec(memory_space=pltpu.VMEM),
        ],
        out_specs=pl.BlockSpec(memory_space=pltpu.VMEM),
        scratch_shapes=[
            pltpu.VMEM((m_per, k), jnp.bfloat16),
            pltpu.VMEM((m_per, k), jnp.bfloat16),
            pltpu.VMEM((m_per, k), jnp.bfloat16),
            pltpu.SemaphoreType.DMA((6,)),
            pltpu.SemaphoreType.DMA((6,)),
        ],
        compiler_params=pltpu.CompilerParams(
            collective_id=0,
            vmem_limit_bytes=100 * 1024 * 1024,
        ),
    )(a_bf, b_bf)


# device time: 126029 ns/iter; 1.5710x vs baseline; 1.0069x over previous
import jax
import jax.numpy as jnp
from jax import lax
from jax.experimental import pallas as pl
from jax.experimental.pallas import tpu as pltpu

N_DEV = 4


def kernel(A, B):
    m_per, k = A.shape
    _, n = B.shape
    half = m_per // 2
    quart = m_per // 4

    a_bf = A.astype(jnp.bfloat16)
    b_bf = B.astype(jnp.bfloat16)

    def body(abf_ref, bbf_ref, out_ref,
             cl_ref, cr_ref, co_ref, send_sems, recv_sems):
        my = lax.axis_index("i")
        left = lax.rem(my + N_DEV - 1, N_DEV)
        right = lax.rem(my + 1, N_DEV)
        opp = lax.rem(my + 2, N_DEV)

        barrier_sem = pltpu.get_barrier_semaphore()
        for nbr in (left, right):
            pl.semaphore_signal(
                barrier_sem, inc=1,
                device_id=(nbr,), device_id_type=pl.DeviceIdType.MESH,
            )
        pl.semaphore_wait(barrier_sem, 2)

        def rdma(src, dst, sem_idx, target):
            return pltpu.make_async_remote_copy(
                src_ref=src, dst_ref=dst,
                send_sem=send_sems.at[sem_idx], recv_sem=recv_sems.at[sem_idx],
                device_id=(target,), device_id_type=pl.DeviceIdType.MESH,
            )

        lo = pl.ds(0, half)
        hi = pl.ds(half, half)
        q = [pl.ds(i * quart, quart) for i in range(4)]

        r0 = rdma(abf_ref.at[lo], cl_ref.at[lo], 0, right)
        r1 = rdma(cl_ref.at[lo], co_ref.at[lo], 1, right)
        r2 = rdma(abf_ref.at[q[2]], cl_ref.at[q[2]], 2, right)
        r3 = rdma(abf_ref.at[q[3]], cl_ref.at[q[3]], 3, right)
        l0 = rdma(abf_ref.at[hi], cr_ref.at[hi], 4, left)
        l1 = rdma(cr_ref.at[hi], co_ref.at[hi], 5, left)
        l2 = rdma(abf_ref.at[q[1]], cr_ref.at[q[1]], 6, left)
        l3 = rdma(abf_ref.at[q[0]], cr_ref.at[q[0]], 7, left)

        r0.start()
        l0.start()

        def mm(chunk, row0, nrows):
            out_ref[pl.ds(row0, nrows), :] = jnp.dot(
                chunk, bbf_ref[...], preferred_element_type=jnp.float32
            ).astype(jnp.bfloat16)

        mm(abf_ref[...], my * m_per, m_per)

        r0.wait_recv()
        r1.start()
        r2.start()
        r3.start()
        mm(cl_ref[lo, :], left * m_per, half)

        l0.wait_recv()
        l1.start()
        l2.start()
        l3.start()
        mm(cr_ref[hi, :], right * m_per + half, half)

        r1.wait_recv()
        mm(co_ref[lo, :], opp * m_per, half)

        l1.wait_recv()
        mm(co_ref[hi, :], opp * m_per + half, half)

        r2.wait_recv()
        mm(cl_ref[q[2], :], left * m_per + 2 * quart, quart)

        l2.wait_recv()
        mm(cr_ref[q[1], :], right * m_per + quart, quart)

        r3.wait_recv()
        mm(cl_ref[q[3], :], left * m_per + 3 * quart, quart)

        l3.wait_recv()
        mm(cr_ref[q[0], :], right * m_per, quart)

        for t in (r0, r1, r2, r3, l0, l1, l2, l3):
            t.wait_send()

    return pl.pallas_call(
        body,
        out_shape=jax.ShapeDtypeStruct((N_DEV * m_per, n), jnp.bfloat16),
        in_specs=[
            pl.BlockSpec(memory_space=pltpu.VMEM),
            pl.BlockSpec(memory_space=pltpu.VMEM),
        ],
        out_specs=pl.BlockSpec(memory_space=pltpu.VMEM),
        scratch_shapes=[
            pltpu.VMEM((m_per, k), jnp.bfloat16),
            pltpu.VMEM((m_per, k), jnp.bfloat16),
            pltpu.VMEM((m_per, k), jnp.bfloat16),
            pltpu.SemaphoreType.DMA((8,)),
            pltpu.SemaphoreType.DMA((8,)),
        ],
        compiler_params=pltpu.CompilerParams(
            collective_id=0,
            vmem_limit_bytes=100 * 1024 * 1024,
        ),
    )(a_bf, b_bf)
